# baseline (speedup 1.0000x reference)
import functools

import jax
import jax.numpy as jnp
from jax import lax
from jax.experimental import pallas as pl
from jax.experimental.pallas import tpu as pltpu
from jax.experimental.pallas import tpu_sc as plsc

EPS = 1e-5
B = 4
C = 64
N = 50000
BLK = 2048
NB = (N + BLK - 1) // BLK
CHUNK = 2000
NCHUNK = N // CHUNK
T_PER_CHUNK = CHUNK // 16
NC = 2
NS = 16
NW = NC * NS
ROWS_PER_WORKER = (B * C) // NW


def _sc_gather_max_body(st_hbm, idx_hbm, out_hbm, row_a, row_b,
                        ia, ib, oa0, oa1, ob0, ob1,
                        sem_row, sem_ia, sem_ib, sem_oa, sem_ob):
    wid = lax.axis_index("s") * NC + lax.axis_index("c")
    b = wid // (NW // B)
    cg = wid % (NW // B)

    idx_sets = ((ia, sem_ia), (ib, sem_ib))
    out_sets = ((oa0, oa1, sem_oa), (ob0, ob1, sem_ob))
    iota3 = lax.iota(jnp.int32, 16) * 3

    def start_idx(ci, s):
        bufs = idx_sets[s]
        return [
            pltpu.async_copy(
                idx_hbm.at[pl.ds((b * N + ci * CHUNK) * 3, 3 * CHUNK)],
                bufs[0], bufs[1])
        ]

    def do_pair(pair, _):
        c0 = cg * ROWS_PER_WORKER + 2 * pair
        rw = [pltpu.async_copy(st_hbm.at[pl.ds((b * C + c0) * N, N)],
                               row_a, sem_row),
              pltpu.async_copy(st_hbm.at[pl.ds((b * C + c0 + 1) * N, N)],
                               row_b, sem_row)]
        pending_idx = start_idx(0, 0)
        for h in rw:
            h.wait()
        pending_out = [None, None]
        for ci in range(NCHUNK):
            s = ci % 2
            cur_idx = pending_idx
            if ci + 1 < NCHUNK:
                pending_idx = start_idx(ci + 1, (ci + 1) % 2)
            for h in cur_idx:
                h.wait()
            if pending_out[s] is not None:
                for h in pending_out[s]:
                    h.wait()
            base = ci * CHUNK
            ibuf, _ = idx_sets[s]
            obuf_a, obuf_b, sem_o = out_sets[s]

            @plsc.parallel_loop(0, T_PER_CHUNK, unroll=5)
            def _(t, ibuf=ibuf, obuf_a=obuf_a, obuf_b=obuf_b, base=base):
                off = t * 16
                v0 = iota3 + off * 3
                i0 = plsc.load_gather(ibuf, [v0])
                i1 = plsc.load_gather(ibuf, [v0 + 1])
                i2 = plsc.load_gather(ibuf, [v0 + 2])
                for row, obuf in ((row_a, obuf_a), (row_b, obuf_b)):
                    g = plsc.load_gather(row, [i0])
                    g = jnp.maximum(g, plsc.load_gather(row, [i1]))
                    g = jnp.maximum(g, plsc.load_gather(row, [i2]))
                    g = jnp.maximum(g, row[pl.ds(base + off, 16)])
                    obuf[pl.ds(off, 16)] = g

            pending_out[s] = [
                pltpu.async_copy(
                    obuf_a, out_hbm.at[pl.ds((b * C + c0) * N + base, CHUNK)],
                    sem_o),
                pltpu.async_copy(
                    obuf_b,
                    out_hbm.at[pl.ds((b * C + c0 + 1) * N + base, CHUNK)],
                    sem_o)]
        for po in pending_out:
            if po is not None:
                for h in po:
                    h.wait()
        return 0

    lax.fori_loop(0, ROWS_PER_WORKER // 2, do_pair, 0)


def _sc_gather_max(st_flat, idx_flat):
    mesh = plsc.VectorSubcoreMesh(core_axis_name="c", subcore_axis_name="s")
    fn = pl.kernel(
        _sc_gather_max_body,
        out_type=jax.ShapeDtypeStruct((B * C * N,), jnp.float32),
        mesh=mesh,
        compiler_params=pltpu.CompilerParams(needs_layout_passes=False),
        scratch_types=[
            pltpu.VMEM((N,), jnp.float32),
            pltpu.VMEM((N,), jnp.float32),
            pltpu.VMEM((3 * CHUNK,), jnp.int32),
            pltpu.VMEM((3 * CHUNK,), jnp.int32),
            pltpu.VMEM((CHUNK,), jnp.float32),
            pltpu.VMEM((CHUNK,), jnp.float32),
            pltpu.VMEM((CHUNK,), jnp.float32),
            pltpu.VMEM((CHUNK,), jnp.float32),
            pltpu.SemaphoreType.DMA,
            pltpu.SemaphoreType.DMA,
            pltpu.SemaphoreType.DMA,
            pltpu.SemaphoreType.DMA,
            pltpu.SemaphoreType.DMA,
        ],
    )
    return fn(st_flat, idx_flat)


def _masked_psums(v, ni):
    lane = lax.broadcasted_iota(jnp.int32, (C, BLK), 1)
    valid = (lane + ni * BLK) < N
    vm = jnp.where(valid, v, 0.0)
    acc1 = jnp.zeros((C, 128), jnp.float32)
    acc2 = jnp.zeros((C, 128), jnp.float32)
    for j in range(BLK // 128):
        chunk = vm[:, j * 128:(j + 1) * 128]
        acc1 = acc1 + chunk
        acc2 = acc2 + chunk * chunk
    return acc1, acc2


def _k1_body(sp_ref, st_ref, w1_ref, w2_ref, b_ref, y_ref, psum_ref):
    bi = pl.program_id(0)
    ni = pl.program_id(1)
    y = jnp.dot(w1_ref[...], sp_ref[0], preferred_element_type=jnp.float32)
    y = y + jnp.dot(w2_ref[...], st_ref[0], preferred_element_type=jnp.float32)
    y = y + b_ref[...]
    y_ref[0] = y.astype(jnp.bfloat16)

    @pl.when((bi == 0) & (ni == 0))
    def _():
        psum_ref[...] = jnp.zeros_like(psum_ref)

    acc1, acc2 = _masked_psums(y, ni)
    psum_ref[0] += acc1
    psum_ref[1] += acc2


def _k1(spatial, structural, w1, w2, bias):
    return pl.pallas_call(
        _k1_body,
        grid=(B, NB),
        in_specs=[
            pl.BlockSpec((1, C, BLK), lambda b, n: (b, 0, n)),
            pl.BlockSpec((1, C, BLK), lambda b, n: (b, 0, n)),
            pl.BlockSpec((C, C), lambda b, n: (0, 0)),
            pl.BlockSpec((C, C), lambda b, n: (0, 0)),
            pl.BlockSpec((C, 1), lambda b, n: (0, 0)),
        ],
        out_specs=[
            pl.BlockSpec((1, C, BLK), lambda b, n: (b, 0, n)),
            pl.BlockSpec((2, C, 128), lambda b, n: (0, 0, 0)),
        ],
        out_shape=[
            jax.ShapeDtypeStruct((B, C, N), jnp.bfloat16),
            jax.ShapeDtypeStruct((2, C, 128), jnp.float32),
        ],
    )(spatial, structural, w1, w2, bias)


def _k2_body(s_ref, w_ref, b_ref, psum_ref):
    bi = pl.program_id(0)
    ni = pl.program_id(1)
    z = jnp.dot(w_ref[...], s_ref[0], preferred_element_type=jnp.float32)
    z = z + b_ref[...]

    @pl.when((bi == 0) & (ni == 0))
    def _():
        psum_ref[...] = jnp.zeros_like(psum_ref)

    acc1, acc2 = _masked_psums(z, ni)
    psum_ref[0] += acc1
    psum_ref[1] += acc2


def _k2(s, w, bias):
    return pl.pallas_call(
        _k2_body,
        grid=(B, NB),
        in_specs=[
            pl.BlockSpec((1, C, BLK), lambda b, n: (b, 0, n)),
            pl.BlockSpec((C, C), lambda b, n: (0, 0)),
            pl.BlockSpec((C, 1), lambda b, n: (0, 0)),
        ],
        out_specs=pl.BlockSpec((2, C, 128), lambda b, n: (0, 0, 0)),
        out_shape=jax.ShapeDtypeStruct((2, C, 128), jnp.float32),
    )(s, w, bias)


def _k3_body(y_ref, s_ref, sy_ref, oy_ref, wz_ref, bz_ref, out1_ref, out2_ref):
    y32 = y_ref[0].astype(jnp.float32)
    out1_ref[0] = jnp.maximum(y32 * sy_ref[...] + oy_ref[...], 0.0)
    z = jnp.dot(wz_ref[...], s_ref[0], preferred_element_type=jnp.float32)
    out2_ref[0] = jnp.maximum(z + bz_ref[...], 0.0)


def _k3(y, s, sy, oy, wz, bz):
    return pl.pallas_call(
        _k3_body,
        grid=(B, NB),
        in_specs=[
            pl.BlockSpec((1, C, BLK), lambda b, n: (b, 0, n)),
            pl.BlockSpec((1, C, BLK), lambda b, n: (b, 0, n)),
            pl.BlockSpec((C, 1), lambda b, n: (0, 0)),
            pl.BlockSpec((C, 1), lambda b, n: (0, 0)),
            pl.BlockSpec((C, C), lambda b, n: (0, 0)),
            pl.BlockSpec((C, 1), lambda b, n: (0, 0)),
        ],
        out_specs=[
            pl.BlockSpec((1, C, BLK), lambda b, n: (b, 0, n)),
            pl.BlockSpec((1, C, BLK), lambda b, n: (b, 0, n)),
        ],
        out_shape=[
            jax.ShapeDtypeStruct((B, C, N), jnp.float32),
            jax.ShapeDtypeStruct((B, C, N), jnp.float32),
        ],
    )(y, s, sy, oy, wz, bz)


def _bn_scale_shift(psum, gamma, beta):
    total = psum.sum(axis=2)
    cnt = float(B * N)
    mean = total[0] / cnt
    var = total[1] / cnt - mean * mean
    scale = gamma * lax.rsqrt(var + EPS)
    shift = beta - mean * scale
    return scale.reshape(C, 1), shift.reshape(C, 1)


def kernel(spatial_feat, structural_feat, neighbor_idx, comb_W, comb_b,
           comb_gamma, comb_beta, agg_W, agg_b, agg_gamma, agg_beta):
    st_flat = structural_feat.reshape(-1)
    idx_flat = neighbor_idx.reshape(-1).astype(jnp.int32)

    s_flat = _sc_gather_max(st_flat, idx_flat)
    s = s_flat.reshape(B, C, N)

    w1 = comb_W[:, :C]
    w2 = comb_W[:, C:]
    y, psum_y = _k1(spatial_feat, structural_feat, w1, w2, comb_b.reshape(C, 1))
    psum_z = _k2(s, agg_W, agg_b.reshape(C, 1))

    sy, oy = _bn_scale_shift(psum_y, comb_gamma, comb_beta)
    sz, oz = _bn_scale_shift(psum_z, agg_gamma, agg_beta)
    wz = agg_W * sz
    bz = agg_b.reshape(C, 1) * sz + oz

    out1, out2 = _k3(y, s, sy, oy, wz, bz)
    return (out1, out2)


# speedup vs baseline: 1.2430x; 1.2430x over previous
import functools

import jax
import jax.numpy as jnp
from jax import lax
from jax.experimental import pallas as pl
from jax.experimental.pallas import tpu as pltpu
from jax.experimental.pallas import tpu_sc as plsc

EPS = 1e-5
B = 4
C = 64
N = 50000
BLK = 2048
NB = (N + BLK - 1) // BLK
CHUNK = 2000
NCHUNK = N // CHUNK
T_PER_CHUNK = CHUNK // 16
NC = 2
NS = 16
NW = NC * NS
ROWS_PER_WORKER = (B * C) // NW


def _sc_gather_max_body(st_hbm, idx_hbm, out_hbm, row_a, row_b,
                        ia, ib, oa0, oa1, ob0, ob1,
                        sem_row, sem_ia, sem_ib, sem_oa, sem_ob):
    wid = lax.axis_index("s") * NC + lax.axis_index("c")
    b = wid // (NW // B)
    cg = wid % (NW // B)

    idx_sets = ((ia, sem_ia), (ib, sem_ib))
    out_sets = ((oa0, oa1, sem_oa), (ob0, ob1, sem_ob))

    def start_idx(ci, s):
        buf, sem = idx_sets[s]
        return [
            pltpu.async_copy(
                idx_hbm.at[pl.ds((b * 3 + k) * N + ci * CHUNK, CHUNK)],
                buf.at[pl.ds(k * CHUNK, CHUNK)], sem)
            for k in range(3)
        ]

    def do_pair(pair, _):
        c0 = cg * ROWS_PER_WORKER + 2 * pair
        rw = [pltpu.async_copy(st_hbm.at[pl.ds((b * C + c0) * N, N)],
                               row_a, sem_row),
              pltpu.async_copy(st_hbm.at[pl.ds((b * C + c0 + 1) * N, N)],
                               row_b, sem_row)]
        pending_idx = start_idx(0, 0)
        for h in rw:
            h.wait()
        pending_out = [None, None]
        for ci in range(NCHUNK):
            s = ci % 2
            cur_idx = pending_idx
            if ci + 1 < NCHUNK:
                pending_idx = start_idx(ci + 1, (ci + 1) % 2)
            for h in cur_idx:
                h.wait()
            if pending_out[s] is not None:
                for h in pending_out[s]:
                    h.wait()
            base = ci * CHUNK
            ibuf, _ = idx_sets[s]
            obuf_a, obuf_b, sem_o = out_sets[s]

            @plsc.parallel_loop(0, T_PER_CHUNK, unroll=5)
            def _(t, ibuf=ibuf, obuf_a=obuf_a, obuf_b=obuf_b, base=base):
                off = t * 16
                i0 = ibuf[pl.ds(off, 16)]
                i1 = ibuf[pl.ds(CHUNK + off, 16)]
                i2 = ibuf[pl.ds(2 * CHUNK + off, 16)]
                for row, obuf in ((row_a, obuf_a), (row_b, obuf_b)):
                    g = plsc.load_gather(row, [i0])
                    g = jnp.maximum(g, plsc.load_gather(row, [i1]))
                    g = jnp.maximum(g, plsc.load_gather(row, [i2]))
                    g = jnp.maximum(g, row[pl.ds(base + off, 16)])
                    obuf[pl.ds(off, 16)] = g

            pending_out[s] = [
                pltpu.async_copy(
                    obuf_a,
                    out_hbm.at[pl.ds((b * C + c0) * N + base, CHUNK)], sem_o),
                pltpu.async_copy(
                    obuf_b,
                    out_hbm.at[pl.ds((b * C + c0 + 1) * N + base, CHUNK)],
                    sem_o)]
        for po in pending_out:
            if po is not None:
                for h in po:
                    h.wait()
        return 0

    lax.fori_loop(0, ROWS_PER_WORKER // 2, do_pair, 0)


def _sc_gather_max(st_flat, idx_flat):
    mesh = plsc.VectorSubcoreMesh(core_axis_name="c", subcore_axis_name="s")
    fn = pl.kernel(
        _sc_gather_max_body,
        out_type=jax.ShapeDtypeStruct((B * C * N,), jnp.float32),
        mesh=mesh,
        compiler_params=pltpu.CompilerParams(needs_layout_passes=False),
        scratch_types=[
            pltpu.VMEM((N,), jnp.float32),
            pltpu.VMEM((N,), jnp.float32),
            pltpu.VMEM((3 * CHUNK,), jnp.int32),
            pltpu.VMEM((3 * CHUNK,), jnp.int32),
            pltpu.VMEM((CHUNK,), jnp.float32),
            pltpu.VMEM((CHUNK,), jnp.float32),
            pltpu.VMEM((CHUNK,), jnp.float32),
            pltpu.VMEM((CHUNK,), jnp.float32),
            pltpu.SemaphoreType.DMA,
            pltpu.SemaphoreType.DMA,
            pltpu.SemaphoreType.DMA,
            pltpu.SemaphoreType.DMA,
            pltpu.SemaphoreType.DMA,
        ],
    )
    return fn(st_flat, idx_flat)


def _masked_psums(v, ni):
    lane = lax.broadcasted_iota(jnp.int32, (C, BLK), 1)
    valid = (lane + ni * BLK) < N
    vm = jnp.where(valid, v, 0.0)
    acc1 = jnp.zeros((C, 128), jnp.float32)
    acc2 = jnp.zeros((C, 128), jnp.float32)
    for j in range(BLK // 128):
        chunk = vm[:, j * 128:(j + 1) * 128]
        acc1 = acc1 + chunk
        acc2 = acc2 + chunk * chunk
    return acc1, acc2


def _k1_body(sp_ref, st_ref, w1_ref, w2_ref, b_ref, y_ref, psum_ref):
    bi = pl.program_id(0)
    ni = pl.program_id(1)
    y = jnp.dot(w1_ref[...], sp_ref[0], preferred_element_type=jnp.float32)
    y = y + jnp.dot(w2_ref[...], st_ref[0], preferred_element_type=jnp.float32)
    y = y + b_ref[...]
    y_ref[0] = y.astype(jnp.bfloat16)

    @pl.when((bi == 0) & (ni == 0))
    def _():
        psum_ref[...] = jnp.zeros_like(psum_ref)

    acc1, acc2 = _masked_psums(y, ni)
    psum_ref[0] += acc1
    psum_ref[1] += acc2


def _k1(spatial, structural, w1, w2, bias):
    return pl.pallas_call(
        _k1_body,
        grid=(B, NB),
        in_specs=[
            pl.BlockSpec((1, C, BLK), lambda b, n: (b, 0, n)),
            pl.BlockSpec((1, C, BLK), lambda b, n: (b, 0, n)),
            pl.BlockSpec((C, C), lambda b, n: (0, 0)),
            pl.BlockSpec((C, C), lambda b, n: (0, 0)),
            pl.BlockSpec((C, 1), lambda b, n: (0, 0)),
        ],
        out_specs=[
            pl.BlockSpec((1, C, BLK), lambda b, n: (b, 0, n)),
            pl.BlockSpec((2, C, 128), lambda b, n: (0, 0, 0)),
        ],
        out_shape=[
            jax.ShapeDtypeStruct((B, C, N), jnp.bfloat16),
            jax.ShapeDtypeStruct((2, C, 128), jnp.float32),
        ],
    )(spatial, structural, w1, w2, bias)


def _k2_body(s_ref, w_ref, b_ref, psum_ref):
    bi = pl.program_id(0)
    ni = pl.program_id(1)
    z = jnp.dot(w_ref[...], s_ref[0], preferred_element_type=jnp.float32)
    z = z + b_ref[...]

    @pl.when((bi == 0) & (ni == 0))
    def _():
        psum_ref[...] = jnp.zeros_like(psum_ref)

    acc1, acc2 = _masked_psums(z, ni)
    psum_ref[0] += acc1
    psum_ref[1] += acc2


def _k2(s, w, bias):
    return pl.pallas_call(
        _k2_body,
        grid=(B, NB),
        in_specs=[
            pl.BlockSpec((1, C, BLK), lambda b, n: (b, 0, n)),
            pl.BlockSpec((C, C), lambda b, n: (0, 0)),
            pl.BlockSpec((C, 1), lambda b, n: (0, 0)),
        ],
        out_specs=pl.BlockSpec((2, C, 128), lambda b, n: (0, 0, 0)),
        out_shape=jax.ShapeDtypeStruct((2, C, 128), jnp.float32),
    )(s, w, bias)


def _k3_body(y_ref, s_ref, sy_ref, oy_ref, wz_ref, bz_ref, out1_ref, out2_ref):
    y32 = y_ref[0].astype(jnp.float32)
    out1_ref[0] = jnp.maximum(y32 * sy_ref[...] + oy_ref[...], 0.0)
    z = jnp.dot(wz_ref[...], s_ref[0], preferred_element_type=jnp.float32)
    out2_ref[0] = jnp.maximum(z + bz_ref[...], 0.0)


def _k3(y, s, sy, oy, wz, bz):
    return pl.pallas_call(
        _k3_body,
        grid=(B, NB),
        in_specs=[
            pl.BlockSpec((1, C, BLK), lambda b, n: (b, 0, n)),
            pl.BlockSpec((1, C, BLK), lambda b, n: (b, 0, n)),
            pl.BlockSpec((C, 1), lambda b, n: (0, 0)),
            pl.BlockSpec((C, 1), lambda b, n: (0, 0)),
            pl.BlockSpec((C, C), lambda b, n: (0, 0)),
            pl.BlockSpec((C, 1), lambda b, n: (0, 0)),
        ],
        out_specs=[
            pl.BlockSpec((1, C, BLK), lambda b, n: (b, 0, n)),
            pl.BlockSpec((1, C, BLK), lambda b, n: (b, 0, n)),
        ],
        out_shape=[
            jax.ShapeDtypeStruct((B, C, N), jnp.float32),
            jax.ShapeDtypeStruct((B, C, N), jnp.float32),
        ],
    )(y, s, sy, oy, wz, bz)


def _bn_scale_shift(psum, gamma, beta):
    total = psum.sum(axis=2)
    cnt = float(B * N)
    mean = total[0] / cnt
    var = total[1] / cnt - mean * mean
    scale = gamma * lax.rsqrt(var + EPS)
    shift = beta - mean * scale
    return scale.reshape(C, 1), shift.reshape(C, 1)


def kernel(spatial_feat, structural_feat, neighbor_idx, comb_W, comb_b,
           comb_gamma, comb_beta, agg_W, agg_b, agg_gamma, agg_beta):
    st_flat = structural_feat.reshape(-1)
    idx_flat = jnp.transpose(neighbor_idx, (0, 2, 1)).reshape(-1)

    s_flat = _sc_gather_max(st_flat, idx_flat)
    s = s_flat.reshape(B, C, N)

    w1 = comb_W[:, :C]
    w2 = comb_W[:, C:]
    y, psum_y = _k1(spatial_feat, structural_feat, w1, w2, comb_b.reshape(C, 1))
    psum_z = _k2(s, agg_W, agg_b.reshape(C, 1))

    sy, oy = _bn_scale_shift(psum_y, comb_gamma, comb_beta)
    sz, oz = _bn_scale_shift(psum_z, agg_gamma, agg_beta)
    wz = agg_W * sz
    bz = agg_b.reshape(C, 1) * sz + oz

    out1, out2 = _k3(y, s, sy, oy, wz, bz)
    return (out1, out2)


# speedup vs baseline: 1.3527x; 1.0882x over previous
import functools

import jax
import jax.numpy as jnp
from jax import lax
from jax.experimental import pallas as pl
from jax.experimental.pallas import tpu as pltpu
from jax.experimental.pallas import tpu_sc as plsc

EPS = 1e-5
B = 4
C = 64
N = 50000
BLK = 4096
NB = (N + BLK - 1) // BLK
CHUNK = 2000
NCHUNK = N // CHUNK
T_PER_CHUNK = CHUNK // 16
NC = 2
NS = 16
NW = NC * NS
ROWS_PER_WORKER = (B * C) // NW


def _sc_gather_max_body(st_hbm, idx_hbm, out_hbm, row_a, row_b,
                        ia, ib, oa0, oa1, ob0, ob1,
                        sem_row, sem_ia, sem_ib, sem_oa, sem_ob):
    wid = lax.axis_index("s") * NC + lax.axis_index("c")
    b = wid // (NW // B)
    cg = wid % (NW // B)

    idx_sets = ((ia, sem_ia), (ib, sem_ib))
    out_sets = ((oa0, oa1, sem_oa), (ob0, ob1, sem_ob))

    def start_idx(ci, s):
        buf, sem = idx_sets[s]
        return [
            pltpu.async_copy(
                idx_hbm.at[pl.ds((b * 3 + k) * N + ci * CHUNK, CHUNK)],
                buf.at[pl.ds(k * CHUNK, CHUNK)], sem)
            for k in range(3)
        ]

    def do_pair(pair, _):
        c0 = cg * ROWS_PER_WORKER + 2 * pair
        rw = [pltpu.async_copy(st_hbm.at[pl.ds((b * C + c0) * N, N)],
                               row_a, sem_row),
              pltpu.async_copy(st_hbm.at[pl.ds((b * C + c0 + 1) * N, N)],
                               row_b, sem_row)]
        pending_idx = start_idx(0, 0)
        for h in rw:
            h.wait()
        pending_out = [None, None]
        for ci in range(NCHUNK):
            s = ci % 2
            cur_idx = pending_idx
            if ci + 1 < NCHUNK:
                pending_idx = start_idx(ci + 1, (ci + 1) % 2)
            for h in cur_idx:
                h.wait()
            if pending_out[s] is not None:
                for h in pending_out[s]:
                    h.wait()
            base = ci * CHUNK
            ibuf, _ = idx_sets[s]
            obuf_a, obuf_b, sem_o = out_sets[s]

            @plsc.parallel_loop(0, T_PER_CHUNK, unroll=8)
            def _(t, ibuf=ibuf, obuf_a=obuf_a, obuf_b=obuf_b, base=base):
                off = t * 16
                i0 = ibuf[pl.ds(off, 16)]
                i1 = ibuf[pl.ds(CHUNK + off, 16)]
                i2 = ibuf[pl.ds(2 * CHUNK + off, 16)]
                for row, obuf in ((row_a, obuf_a), (row_b, obuf_b)):
                    g = plsc.load_gather(row, [i0])
                    g = jnp.maximum(g, plsc.load_gather(row, [i1]))
                    g = jnp.maximum(g, plsc.load_gather(row, [i2]))
                    g = jnp.maximum(g, row[pl.ds(base + off, 16)])
                    obuf[pl.ds(off, 16)] = g

            pending_out[s] = [
                pltpu.async_copy(
                    obuf_a,
                    out_hbm.at[pl.ds((b * C + c0) * N + base, CHUNK)], sem_o),
                pltpu.async_copy(
                    obuf_b,
                    out_hbm.at[pl.ds((b * C + c0 + 1) * N + base, CHUNK)],
                    sem_o)]
        for po in pending_out:
            if po is not None:
                for h in po:
                    h.wait()
        return 0

    lax.fori_loop(0, ROWS_PER_WORKER // 2, do_pair, 0)


def _sc_gather_max(st_flat, idx_flat):
    mesh = plsc.VectorSubcoreMesh(core_axis_name="c", subcore_axis_name="s")
    fn = pl.kernel(
        _sc_gather_max_body,
        out_type=jax.ShapeDtypeStruct((B * C * N,), jnp.float32),
        mesh=mesh,
        compiler_params=pltpu.CompilerParams(needs_layout_passes=False),
        scratch_types=[
            pltpu.VMEM((N,), jnp.float32),
            pltpu.VMEM((N,), jnp.float32),
            pltpu.VMEM((3 * CHUNK,), jnp.int32),
            pltpu.VMEM((3 * CHUNK,), jnp.int32),
            pltpu.VMEM((CHUNK,), jnp.float32),
            pltpu.VMEM((CHUNK,), jnp.float32),
            pltpu.VMEM((CHUNK,), jnp.float32),
            pltpu.VMEM((CHUNK,), jnp.float32),
            pltpu.SemaphoreType.DMA,
            pltpu.SemaphoreType.DMA,
            pltpu.SemaphoreType.DMA,
            pltpu.SemaphoreType.DMA,
            pltpu.SemaphoreType.DMA,
        ],
    )
    return fn(st_flat, idx_flat)


def _masked_psums(v, ni):
    lane = lax.broadcasted_iota(jnp.int32, (C, BLK), 1)
    valid = (lane + ni * BLK) < N
    vm = jnp.where(valid, v, 0.0)
    acc1 = jnp.zeros((C, 128), jnp.float32)
    acc2 = jnp.zeros((C, 128), jnp.float32)
    for j in range(BLK // 128):
        chunk = vm[:, j * 128:(j + 1) * 128]
        acc1 = acc1 + chunk
        acc2 = acc2 + chunk * chunk
    return acc1, acc2


def _k1_body(sp_ref, st_ref, w1_ref, w2_ref, b_ref, y_ref, psum_ref):
    bi = pl.program_id(0)
    ni = pl.program_id(1)
    y = jnp.dot(w1_ref[...], sp_ref[0], preferred_element_type=jnp.float32)
    y = y + jnp.dot(w2_ref[...], st_ref[0], preferred_element_type=jnp.float32)
    y = y + b_ref[...]
    y_ref[0] = y.astype(jnp.bfloat16)

    @pl.when((bi == 0) & (ni == 0))
    def _():
        psum_ref[...] = jnp.zeros_like(psum_ref)

    acc1, acc2 = _masked_psums(y, ni)
    psum_ref[0] += acc1
    psum_ref[1] += acc2


def _k1(spatial, structural, w1, w2, bias):
    return pl.pallas_call(
        _k1_body,
        grid=(B, NB),
        in_specs=[
            pl.BlockSpec((1, C, BLK), lambda b, n: (b, 0, n)),
            pl.BlockSpec((1, C, BLK), lambda b, n: (b, 0, n)),
            pl.BlockSpec((C, C), lambda b, n: (0, 0)),
            pl.BlockSpec((C, C), lambda b, n: (0, 0)),
            pl.BlockSpec((C, 1), lambda b, n: (0, 0)),
        ],
        out_specs=[
            pl.BlockSpec((1, C, BLK), lambda b, n: (b, 0, n)),
            pl.BlockSpec((2, C, 128), lambda b, n: (0, 0, 0)),
        ],
        out_shape=[
            jax.ShapeDtypeStruct((B, C, N), jnp.bfloat16),
            jax.ShapeDtypeStruct((2, C, 128), jnp.float32),
        ],
    )(spatial, structural, w1, w2, bias)


def _k2_body(s_ref, w_ref, b_ref, psum_ref):
    bi = pl.program_id(0)
    ni = pl.program_id(1)
    z = jnp.dot(w_ref[...], s_ref[0], preferred_element_type=jnp.float32)
    z = z + b_ref[...]

    @pl.when((bi == 0) & (ni == 0))
    def _():
        psum_ref[...] = jnp.zeros_like(psum_ref)

    acc1, acc2 = _masked_psums(z, ni)
    psum_ref[0] += acc1
    psum_ref[1] += acc2


def _k2(s, w, bias):
    return pl.pallas_call(
        _k2_body,
        grid=(B, NB),
        in_specs=[
            pl.BlockSpec((1, C, BLK), lambda b, n: (b, 0, n)),
            pl.BlockSpec((C, C), lambda b, n: (0, 0)),
            pl.BlockSpec((C, 1), lambda b, n: (0, 0)),
        ],
        out_specs=pl.BlockSpec((2, C, 128), lambda b, n: (0, 0, 0)),
        out_shape=jax.ShapeDtypeStruct((2, C, 128), jnp.float32),
    )(s, w, bias)


def _k3_body(y_ref, s_ref, sy_ref, oy_ref, wz_ref, bz_ref, out1_ref, out2_ref):
    y32 = y_ref[0].astype(jnp.float32)
    out1_ref[0] = jnp.maximum(y32 * sy_ref[...] + oy_ref[...], 0.0)
    z = jnp.dot(wz_ref[...], s_ref[0], preferred_element_type=jnp.float32)
    out2_ref[0] = jnp.maximum(z + bz_ref[...], 0.0)


def _k3(y, s, sy, oy, wz, bz):
    return pl.pallas_call(
        _k3_body,
        grid=(B, NB),
        in_specs=[
            pl.BlockSpec((1, C, BLK), lambda b, n: (b, 0, n)),
            pl.BlockSpec((1, C, BLK), lambda b, n: (b, 0, n)),
            pl.BlockSpec((C, 1), lambda b, n: (0, 0)),
            pl.BlockSpec((C, 1), lambda b, n: (0, 0)),
            pl.BlockSpec((C, C), lambda b, n: (0, 0)),
            pl.BlockSpec((C, 1), lambda b, n: (0, 0)),
        ],
        out_specs=[
            pl.BlockSpec((1, C, BLK), lambda b, n: (b, 0, n)),
            pl.BlockSpec((1, C, BLK), lambda b, n: (b, 0, n)),
        ],
        out_shape=[
            jax.ShapeDtypeStruct((B, C, N), jnp.float32),
            jax.ShapeDtypeStruct((B, C, N), jnp.float32),
        ],
    )(y, s, sy, oy, wz, bz)


def _bn_scale_shift(psum, gamma, beta):
    total = psum.sum(axis=2)
    cnt = float(B * N)
    mean = total[0] / cnt
    var = total[1] / cnt - mean * mean
    scale = gamma * lax.rsqrt(var + EPS)
    shift = beta - mean * scale
    return scale.reshape(C, 1), shift.reshape(C, 1)


def kernel(spatial_feat, structural_feat, neighbor_idx, comb_W, comb_b,
           comb_gamma, comb_beta, agg_W, agg_b, agg_gamma, agg_beta):
    st_flat = structural_feat.reshape(-1)
    idx_flat = jnp.transpose(neighbor_idx, (0, 2, 1)).reshape(-1)

    s_flat = _sc_gather_max(st_flat, idx_flat)
    s = s_flat.reshape(B, C, N)

    w1 = comb_W[:, :C]
    w2 = comb_W[:, C:]
    y, psum_y = _k1(spatial_feat, structural_feat, w1, w2, comb_b.reshape(C, 1))
    psum_z = _k2(s, agg_W, agg_b.reshape(C, 1))

    sy, oy = _bn_scale_shift(psum_y, comb_gamma, comb_beta)
    sz, oz = _bn_scale_shift(psum_z, agg_gamma, agg_beta)
    wz = agg_W * sz
    bz = agg_b.reshape(C, 1) * sz + oz

    out1, out2 = _k3(y, s, sy, oy, wz, bz)
    return (out1, out2)


# speedup vs baseline: 1.4572x; 1.0773x over previous
import functools

import jax
import jax.numpy as jnp
from jax import lax
from jax.experimental import pallas as pl
from jax.experimental.pallas import tpu as pltpu
from jax.experimental.pallas import tpu_sc as plsc

EPS = 1e-5
B = 4
C = 64
N = 50000
BLK = 4096
NB = (N + BLK - 1) // BLK
NPAD = 50176
CHUNK = 2048
NCHUNK = 25
TAIL = N - 24 * CHUNK
TAIL_DMA = 1024
NC = 2
NS = 16
NW = NC * NS
ROWS_PER_WORKER = (B * C) // NW


def _sc_gather_max_body(st_hbm, idx_hbm, out_hbm, row_a, row_b,
                        ia, ib, oa0, oa1, ob0, ob1,
                        sem_row, sem_ia, sem_ib, sem_oa, sem_ob):
    wid = lax.axis_index("s") * NC + lax.axis_index("c")
    b = wid // (NW // B)
    cg = wid % (NW // B)

    idx_sets = ((ia, sem_ia), (ib, sem_ib))
    out_sets = ((oa0, oa1, sem_oa), (ob0, ob1, sem_ob))

    def start_idx(ci, s):
        buf, sem = idx_sets[s]
        ln = CHUNK if ci < NCHUNK - 1 else TAIL
        return [
            pltpu.async_copy(
                idx_hbm.at[pl.ds((b * 3 + k) * N + ci * CHUNK, ln)],
                buf.at[pl.ds(k * CHUNK, ln)], sem)
            for k in range(3)
        ]

    def do_pair(pair, _):
        c0 = cg * ROWS_PER_WORKER + 2 * pair
        rw = [pltpu.async_copy(st_hbm.at[pl.ds((b * C + c0) * N, N)],
                               row_a, sem_row),
              pltpu.async_copy(st_hbm.at[pl.ds((b * C + c0 + 1) * N, N)],
                               row_b, sem_row)]
        pending_idx = start_idx(0, 0)
        for h in rw:
            h.wait()
        pending_out = [None, None]
        for ci in range(NCHUNK):
            s = ci % 2
            cur_idx = pending_idx
            if ci + 1 < NCHUNK:
                pending_idx = start_idx(ci + 1, (ci + 1) % 2)
            for h in cur_idx:
                h.wait()
            if pending_out[s] is not None:
                for h in pending_out[s]:
                    h.wait()
            base = ci * CHUNK
            ibuf, _ = idx_sets[s]
            obuf_a, obuf_b, sem_o = out_sets[s]

            def gather16(row, off):
                i0 = ibuf[pl.ds(off, 16)]
                i1 = ibuf[pl.ds(CHUNK + off, 16)]
                i2 = ibuf[pl.ds(2 * CHUNK + off, 16)]
                g = plsc.load_gather(row, [i0])
                g = jnp.maximum(g, plsc.load_gather(row, [i1]))
                g = jnp.maximum(g, plsc.load_gather(row, [i2]))
                return jnp.maximum(g, row[pl.ds(base + off, 16)])

            n32 = (CHUNK if ci < NCHUNK - 1 else TAIL - 16) // 32

            @plsc.parallel_loop(0, n32, unroll=4)
            def _(t, gather16=gather16, obuf_a=obuf_a, obuf_b=obuf_b):
                off = t * 32
                for row, obuf in ((row_a, obuf_a), (row_b, obuf_b)):
                    g_lo = gather16(row, off)
                    g_hi = gather16(row, off + 16)
                    obuf[pl.ds(off, 32)] = plsc.pack(
                        g_lo, g_hi, format=plsc.PackFormat.INTERLEAVED)

            if ci == NCHUNK - 1:
                for row, obuf in ((row_a, obuf_a), (row_b, obuf_b)):
                    g_lo = gather16(row, TAIL - 16)
                    obuf[pl.ds(TAIL - 16, 32)] = plsc.pack(
                        g_lo, g_lo, format=plsc.PackFormat.INTERLEAVED)

            ln = CHUNK if ci < NCHUNK - 1 else TAIL_DMA
            pending_out[s] = [
                pltpu.async_copy(
                    obuf_a.at[pl.ds(0, ln)],
                    out_hbm.at[pl.ds((b * C + c0) * NPAD + base, ln)], sem_o),
                pltpu.async_copy(
                    obuf_b.at[pl.ds(0, ln)],
                    out_hbm.at[pl.ds((b * C + c0 + 1) * NPAD + base, ln)],
                    sem_o)]
        for po in pending_out:
            if po is not None:
                for h in po:
                    h.wait()
        return 0

    lax.fori_loop(0, ROWS_PER_WORKER // 2, do_pair, 0)


def _sc_gather_max(st_flat, idx_flat):
    mesh = plsc.VectorSubcoreMesh(core_axis_name="c", subcore_axis_name="s")
    fn = pl.kernel(
        _sc_gather_max_body,
        out_type=jax.ShapeDtypeStruct((B * C * NPAD,), jnp.bfloat16),
        mesh=mesh,
        compiler_params=pltpu.CompilerParams(needs_layout_passes=False),
        scratch_types=[
            pltpu.VMEM((N,), jnp.float32),
            pltpu.VMEM((N,), jnp.float32),
            pltpu.VMEM((3 * CHUNK,), jnp.int32),
            pltpu.VMEM((3 * CHUNK,), jnp.int32),
            pltpu.VMEM((CHUNK + 16,), jnp.bfloat16),
            pltpu.VMEM((CHUNK + 16,), jnp.bfloat16),
            pltpu.VMEM((CHUNK + 16,), jnp.bfloat16),
            pltpu.VMEM((CHUNK + 16,), jnp.bfloat16),
            pltpu.SemaphoreType.DMA,
            pltpu.SemaphoreType.DMA,
            pltpu.SemaphoreType.DMA,
            pltpu.SemaphoreType.DMA,
            pltpu.SemaphoreType.DMA,
        ],
    )
    return fn(st_flat, idx_flat)


def _masked_psums(v, ni):
    lane = lax.broadcasted_iota(jnp.int32, (C, BLK), 1)
    valid = (lane + ni * BLK) < N
    vm = jnp.where(valid, v, 0.0)
    acc1 = jnp.zeros((C, 128), jnp.float32)
    acc2 = jnp.zeros((C, 128), jnp.float32)
    for j in range(BLK // 128):
        chunk = vm[:, j * 128:(j + 1) * 128]
        acc1 = acc1 + chunk
        acc2 = acc2 + chunk * chunk
    return acc1, acc2


def _k1_body(sp_ref, st_ref, w1_ref, w2_ref, b_ref, y_ref, psum_ref):
    bi = pl.program_id(0)
    ni = pl.program_id(1)
    y = jnp.dot(w1_ref[...], sp_ref[0], preferred_element_type=jnp.float32)
    y = y + jnp.dot(w2_ref[...], st_ref[0], preferred_element_type=jnp.float32)
    y = y + b_ref[...]
    y_ref[0] = y.astype(jnp.bfloat16)

    @pl.when((bi == 0) & (ni == 0))
    def _():
        psum_ref[...] = jnp.zeros_like(psum_ref)

    acc1, acc2 = _masked_psums(y, ni)
    psum_ref[0] += acc1
    psum_ref[1] += acc2


def _k1(spatial, structural, w1, w2, bias):
    return pl.pallas_call(
        _k1_body,
        grid=(B, NB),
        in_specs=[
            pl.BlockSpec((1, C, BLK), lambda b, n: (b, 0, n)),
            pl.BlockSpec((1, C, BLK), lambda b, n: (b, 0, n)),
            pl.BlockSpec((C, C), lambda b, n: (0, 0)),
            pl.BlockSpec((C, C), lambda b, n: (0, 0)),
            pl.BlockSpec((C, 1), lambda b, n: (0, 0)),
        ],
        out_specs=[
            pl.BlockSpec((1, C, BLK), lambda b, n: (b, 0, n)),
            pl.BlockSpec((2, C, 128), lambda b, n: (0, 0, 0)),
        ],
        out_shape=[
            jax.ShapeDtypeStruct((B, C, N), jnp.bfloat16),
            jax.ShapeDtypeStruct((2, C, 128), jnp.float32),
        ],
    )(spatial, structural, w1, w2, bias)


def _k2_body(s_ref, w_ref, b_ref, psum_ref):
    bi = pl.program_id(0)
    ni = pl.program_id(1)
    z = jnp.dot(w_ref[...], s_ref[0].astype(jnp.float32),
                preferred_element_type=jnp.float32)
    z = z + b_ref[...]

    @pl.when((bi == 0) & (ni == 0))
    def _():
        psum_ref[...] = jnp.zeros_like(psum_ref)

    acc1, acc2 = _masked_psums(z, ni)
    psum_ref[0] += acc1
    psum_ref[1] += acc2


def _k2(s, w, bias):
    return pl.pallas_call(
        _k2_body,
        grid=(B, NB),
        in_specs=[
            pl.BlockSpec((1, C, BLK), lambda b, n: (b, 0, n)),
            pl.BlockSpec((C, C), lambda b, n: (0, 0)),
            pl.BlockSpec((C, 1), lambda b, n: (0, 0)),
        ],
        out_specs=pl.BlockSpec((2, C, 128), lambda b, n: (0, 0, 0)),
        out_shape=jax.ShapeDtypeStruct((2, C, 128), jnp.float32),
    )(s, w, bias)


def _k3_body(y_ref, s_ref, sy_ref, oy_ref, wz_ref, bz_ref, out1_ref, out2_ref):
    y32 = y_ref[0].astype(jnp.float32)
    out1_ref[0] = jnp.maximum(y32 * sy_ref[...] + oy_ref[...], 0.0)
    z = jnp.dot(wz_ref[...], s_ref[0].astype(jnp.float32),
                preferred_element_type=jnp.float32)
    out2_ref[0] = jnp.maximum(z + bz_ref[...], 0.0)


def _k3(y, s, sy, oy, wz, bz):
    return pl.pallas_call(
        _k3_body,
        grid=(B, NB),
        in_specs=[
            pl.BlockSpec((1, C, BLK), lambda b, n: (b, 0, n)),
            pl.BlockSpec((1, C, BLK), lambda b, n: (b, 0, n)),
            pl.BlockSpec((C, 1), lambda b, n: (0, 0)),
            pl.BlockSpec((C, 1), lambda b, n: (0, 0)),
            pl.BlockSpec((C, C), lambda b, n: (0, 0)),
            pl.BlockSpec((C, 1), lambda b, n: (0, 0)),
        ],
        out_specs=[
            pl.BlockSpec((1, C, BLK), lambda b, n: (b, 0, n)),
            pl.BlockSpec((1, C, BLK), lambda b, n: (b, 0, n)),
        ],
        out_shape=[
            jax.ShapeDtypeStruct((B, C, N), jnp.float32),
            jax.ShapeDtypeStruct((B, C, N), jnp.float32),
        ],
    )(y, s, sy, oy, wz, bz)


def _bn_scale_shift(psum, gamma, beta):
    total = psum.sum(axis=2)
    cnt = float(B * N)
    mean = total[0] / cnt
    var = total[1] / cnt - mean * mean
    scale = gamma * lax.rsqrt(var + EPS)
    shift = beta - mean * scale
    return scale.reshape(C, 1), shift.reshape(C, 1)


def kernel(spatial_feat, structural_feat, neighbor_idx, comb_W, comb_b,
           comb_gamma, comb_beta, agg_W, agg_b, agg_gamma, agg_beta):
    st_flat = structural_feat.reshape(-1)
    idx_flat = jnp.transpose(neighbor_idx, (0, 2, 1)).reshape(-1)

    s_flat = _sc_gather_max(st_flat, idx_flat)
    s = s_flat.reshape(B, C, NPAD)

    w1 = comb_W[:, :C]
    w2 = comb_W[:, C:]
    y, psum_y = _k1(spatial_feat, structural_feat, w1, w2, comb_b.reshape(C, 1))
    psum_z = _k2(s, agg_W, agg_b.reshape(C, 1))

    sy, oy = _bn_scale_shift(psum_y, comb_gamma, comb_beta)
    sz, oz = _bn_scale_shift(psum_z, agg_gamma, agg_beta)
    wz = agg_W * sz
    bz = agg_b.reshape(C, 1) * sz + oz

    out1, out2 = _k3(y, s, sy, oy, wz, bz)
    return (out1, out2)


# speedup vs baseline: 1.4735x; 1.0112x over previous
import functools

import jax
import jax.numpy as jnp
from jax import lax
from jax.experimental import pallas as pl
from jax.experimental.pallas import tpu as pltpu
from jax.experimental.pallas import tpu_sc as plsc

EPS = 1e-5
B = 4
C = 64
N = 50000
BLK = 4096
NB = (N + BLK - 1) // BLK
NPAD = 50176
CHUNK = 2048
NCHUNK = 25
TAIL = N - 24 * CHUNK
TAIL_DMA = 1024
NC = 2
NS = 16
NW = NC * NS
ROWS_PER_WORKER = (B * C) // NW


def _sc_gather_max_body(st_hbm, idx_hbm, out_hbm, row_a, row_b,
                        ia, ib, oa0, oa1, ob0, ob1,
                        sem_row, sem_ia, sem_ib, sem_oa, sem_ob):
    wid = lax.axis_index("s") * NC + lax.axis_index("c")
    b = wid // (NW // B)
    cg = wid % (NW // B)

    idx_sets = ((ia, sem_ia), (ib, sem_ib))
    out_sets = ((oa0, oa1, sem_oa), (ob0, ob1, sem_ob))

    def start_idx(ci, s):
        buf, sem = idx_sets[s]
        ln = CHUNK if ci < NCHUNK - 1 else TAIL
        return [
            pltpu.async_copy(
                idx_hbm.at[pl.ds((b * 3 + k) * N + ci * CHUNK, ln)],
                buf.at[pl.ds(k * CHUNK, ln)], sem)
            for k in range(3)
        ]

    def do_pair(pair, _):
        c0 = cg * ROWS_PER_WORKER + 2 * pair
        rw = [pltpu.async_copy(st_hbm.at[pl.ds((b * C + c0) * N, N)],
                               row_a, sem_row),
              pltpu.async_copy(st_hbm.at[pl.ds((b * C + c0 + 1) * N, N)],
                               row_b, sem_row)]
        pending_idx = start_idx(0, 0)
        for h in rw:
            h.wait()
        pending_out = [None, None]
        for ci in range(NCHUNK):
            s = ci % 2
            cur_idx = pending_idx
            if ci + 1 < NCHUNK:
                pending_idx = start_idx(ci + 1, (ci + 1) % 2)
            for h in cur_idx:
                h.wait()
            if pending_out[s] is not None:
                for h in pending_out[s]:
                    h.wait()
            base = ci * CHUNK
            ibuf, _ = idx_sets[s]
            obuf_a, obuf_b, sem_o = out_sets[s]

            def gather16(row, off):
                i0 = ibuf[pl.ds(off, 16)]
                i1 = ibuf[pl.ds(CHUNK + off, 16)]
                i2 = ibuf[pl.ds(2 * CHUNK + off, 16)]
                g = plsc.load_gather(row, [i0])
                g = jnp.maximum(g, plsc.load_gather(row, [i1]))
                g = jnp.maximum(g, plsc.load_gather(row, [i2]))
                return jnp.maximum(g, row[pl.ds(base + off, 16)])

            n32 = (CHUNK if ci < NCHUNK - 1 else TAIL - 16) // 32

            @plsc.parallel_loop(0, n32, unroll=4)
            def _(t, gather16=gather16, obuf_a=obuf_a, obuf_b=obuf_b):
                off = t * 32
                for row, obuf in ((row_a, obuf_a), (row_b, obuf_b)):
                    g_lo = gather16(row, off)
                    g_hi = gather16(row, off + 16)
                    obuf[pl.ds(off, 32)] = plsc.pack(
                        g_lo, g_hi, format=plsc.PackFormat.INTERLEAVED)

            if ci == NCHUNK - 1:
                for row, obuf in ((row_a, obuf_a), (row_b, obuf_b)):
                    g_lo = gather16(row, TAIL - 16)
                    obuf[pl.ds(TAIL - 16, 32)] = plsc.pack(
                        g_lo, g_lo, format=plsc.PackFormat.INTERLEAVED)

            ln = CHUNK if ci < NCHUNK - 1 else TAIL_DMA
            pending_out[s] = [
                pltpu.async_copy(
                    obuf_a.at[pl.ds(0, ln)],
                    out_hbm.at[pl.ds((b * C + c0) * NPAD + base, ln)], sem_o),
                pltpu.async_copy(
                    obuf_b.at[pl.ds(0, ln)],
                    out_hbm.at[pl.ds((b * C + c0 + 1) * NPAD + base, ln)],
                    sem_o)]
        for po in pending_out:
            if po is not None:
                for h in po:
                    h.wait()
        return 0

    lax.fori_loop(0, ROWS_PER_WORKER // 2, do_pair, 0)


def _sc_gather_max(st_flat, idx_flat):
    mesh = plsc.VectorSubcoreMesh(core_axis_name="c", subcore_axis_name="s")
    fn = pl.kernel(
        _sc_gather_max_body,
        out_type=jax.ShapeDtypeStruct((B * C * NPAD,), jnp.bfloat16),
        mesh=mesh,
        compiler_params=pltpu.CompilerParams(needs_layout_passes=False),
        scratch_types=[
            pltpu.VMEM((N,), jnp.float32),
            pltpu.VMEM((N,), jnp.float32),
            pltpu.VMEM((3 * CHUNK,), jnp.int32),
            pltpu.VMEM((3 * CHUNK,), jnp.int32),
            pltpu.VMEM((CHUNK + 16,), jnp.bfloat16),
            pltpu.VMEM((CHUNK + 16,), jnp.bfloat16),
            pltpu.VMEM((CHUNK + 16,), jnp.bfloat16),
            pltpu.VMEM((CHUNK + 16,), jnp.bfloat16),
            pltpu.SemaphoreType.DMA,
            pltpu.SemaphoreType.DMA,
            pltpu.SemaphoreType.DMA,
            pltpu.SemaphoreType.DMA,
            pltpu.SemaphoreType.DMA,
        ],
    )
    return fn(st_flat, idx_flat)


def _masked_psums(v, ni):
    lane = lax.broadcasted_iota(jnp.int32, (C, BLK), 1)
    valid = (lane + ni * BLK) < N
    vm = jnp.where(valid, v, 0.0)
    acc1 = jnp.zeros((C, 128), jnp.float32)
    acc2 = jnp.zeros((C, 128), jnp.float32)
    for j in range(BLK // 128):
        chunk = vm[:, j * 128:(j + 1) * 128]
        acc1 = acc1 + chunk
        acc2 = acc2 + chunk * chunk
    return acc1, acc2


def _k1_body(sp_ref, st_ref, w1_ref, w2_ref, b_ref, y_ref, psum_ref):
    bi = pl.program_id(0)
    ni = pl.program_id(1)
    y = jnp.dot(w1_ref[...], sp_ref[0], preferred_element_type=jnp.float32)
    y = y + jnp.dot(w2_ref[...], st_ref[0], preferred_element_type=jnp.float32)
    y = y + b_ref[...]
    y_ref[0] = y.astype(jnp.bfloat16)

    @pl.when((bi == 0) & (ni == 0))
    def _():
        psum_ref[...] = jnp.zeros_like(psum_ref)

    acc1, acc2 = _masked_psums(y, ni)
    psum_ref[0] += acc1
    psum_ref[1] += acc2


def _k1(spatial, structural, w1, w2, bias):
    return pl.pallas_call(
        _k1_body,
        grid=(B, NB),
        in_specs=[
            pl.BlockSpec((1, C, BLK), lambda b, n: (b, 0, n)),
            pl.BlockSpec((1, C, BLK), lambda b, n: (b, 0, n)),
            pl.BlockSpec((C, C), lambda b, n: (0, 0)),
            pl.BlockSpec((C, C), lambda b, n: (0, 0)),
            pl.BlockSpec((C, 1), lambda b, n: (0, 0)),
        ],
        out_specs=[
            pl.BlockSpec((1, C, BLK), lambda b, n: (b, 0, n)),
            pl.BlockSpec((2, C, 128), lambda b, n: (0, 0, 0)),
        ],
        out_shape=[
            jax.ShapeDtypeStruct((B, C, N), jnp.bfloat16),
            jax.ShapeDtypeStruct((2, C, 128), jnp.float32),
        ],
    )(spatial, structural, w1, w2, bias)


def _k23_body(y_ref, s_ref, sy_ref, oy_ref, w_ref, b_ref, g_ref, be_ref,
              out1_ref, out2_ref, psum_ref, sz_ref, oz_ref):
    p = pl.program_id(0)
    bi = pl.program_id(1)
    ni = pl.program_id(2)

    @pl.when(p == 0)
    def _():
        @pl.when((bi == 0) & (ni == 0))
        def _():
            psum_ref[...] = jnp.zeros_like(psum_ref)

        z = jnp.dot(w_ref[...], s_ref[0].astype(jnp.float32),
                    preferred_element_type=jnp.float32) + b_ref[...]
        acc1, acc2 = _masked_psums(z, ni)
        psum_ref[0] += acc1
        psum_ref[1] += acc2

    @pl.when(p == 1)
    def _():
        @pl.when((bi == 0) & (ni == 0))
        def _():
            tot = jnp.sum(psum_ref[...], axis=2)
            cnt = float(B * N)
            mean = tot[0] / cnt
            var = tot[1] / cnt - mean * mean
            scale = g_ref[0] * lax.rsqrt(var + EPS)
            shift = be_ref[0] - mean * scale
            sz_ref[...] = scale.reshape(C, 1)
            oz_ref[...] = shift.reshape(C, 1)

        y32 = y_ref[0].astype(jnp.float32)
        out1_ref[0] = jnp.maximum(y32 * sy_ref[...] + oy_ref[...], 0.0)
        z = jnp.dot(w_ref[...], s_ref[0].astype(jnp.float32),
                    preferred_element_type=jnp.float32) + b_ref[...]
        out2_ref[0] = jnp.maximum(z * sz_ref[...] + oz_ref[...], 0.0)


def _k23(y, s, sy, oy, w, bias, gamma, beta):
    return pl.pallas_call(
        _k23_body,
        grid=(2, B, NB),
        in_specs=[
            pl.BlockSpec((1, C, BLK), lambda p, b, n: (b * p, 0, n * p)),
            pl.BlockSpec((1, C, BLK), lambda p, b, n: (b, 0, n)),
            pl.BlockSpec((C, 1), lambda p, b, n: (0, 0)),
            pl.BlockSpec((C, 1), lambda p, b, n: (0, 0)),
            pl.BlockSpec((C, C), lambda p, b, n: (0, 0)),
            pl.BlockSpec((C, 1), lambda p, b, n: (0, 0)),
            pl.BlockSpec((1, C), lambda p, b, n: (0, 0)),
            pl.BlockSpec((1, C), lambda p, b, n: (0, 0)),
        ],
        out_specs=[
            pl.BlockSpec((1, C, BLK), lambda p, b, n: (b * p, 0, n * p)),
            pl.BlockSpec((1, C, BLK), lambda p, b, n: (b * p, 0, n * p)),
        ],
        out_shape=[
            jax.ShapeDtypeStruct((B, C, N), jnp.float32),
            jax.ShapeDtypeStruct((B, C, N), jnp.float32),
        ],
        scratch_shapes=[
            pltpu.VMEM((2, C, 128), jnp.float32),
            pltpu.VMEM((C, 1), jnp.float32),
            pltpu.VMEM((C, 1), jnp.float32),
        ],
    )(y, s, sy, oy, w, bias, gamma, beta)


def _bn_scale_shift(psum, gamma, beta):
    total = psum.sum(axis=2)
    cnt = float(B * N)
    mean = total[0] / cnt
    var = total[1] / cnt - mean * mean
    scale = gamma * lax.rsqrt(var + EPS)
    shift = beta - mean * scale
    return scale.reshape(C, 1), shift.reshape(C, 1)


def kernel(spatial_feat, structural_feat, neighbor_idx, comb_W, comb_b,
           comb_gamma, comb_beta, agg_W, agg_b, agg_gamma, agg_beta):
    st_flat = structural_feat.reshape(-1)
    idx_flat = jnp.transpose(neighbor_idx, (0, 2, 1)).reshape(-1)

    s_flat = _sc_gather_max(st_flat, idx_flat)
    s = s_flat.reshape(B, C, NPAD)

    w1 = comb_W[:, :C]
    w2 = comb_W[:, C:]
    y, psum_y = _k1(spatial_feat, structural_feat, w1, w2, comb_b.reshape(C, 1))

    sy, oy = _bn_scale_shift(psum_y, comb_gamma, comb_beta)

    out1, out2 = _k23(y, s, sy, oy, agg_W, agg_b.reshape(C, 1),
                      agg_gamma.reshape(1, C), agg_beta.reshape(1, C))
    return (out1, out2)


# speedup vs baseline: 1.5580x; 1.0574x over previous
import functools

import jax
import jax.numpy as jnp
from jax import lax
from jax.experimental import pallas as pl
from jax.experimental.pallas import tpu as pltpu
from jax.experimental.pallas import tpu_sc as plsc

EPS = 1e-5
B = 4
C = 64
N = 50000
BLK = 8192
NB = (N + BLK - 1) // BLK
NPAD = 50176
CHUNK = 2048
NCHUNK = 25
TAIL = N - 24 * CHUNK
TAIL_DMA = 1024
NC = 2
NS = 16
NW = NC * NS
ROWS_PER_WORKER = (B * C) // NW


def _sc_gather_max_body(st_hbm, idx_hbm, out_hbm, row_a, row_b,
                        ia, ib, oa0, oa1, ob0, ob1,
                        sem_row, sem_ia, sem_ib, sem_oa, sem_ob):
    wid = lax.axis_index("s") * NC + lax.axis_index("c")
    b = wid // (NW // B)
    cg = wid % (NW // B)

    idx_sets = ((ia, sem_ia), (ib, sem_ib))
    out_sets = ((oa0, oa1, sem_oa), (ob0, ob1, sem_ob))

    def start_idx(ci, s):
        buf, sem = idx_sets[s]
        ln = CHUNK if ci < NCHUNK - 1 else TAIL
        return [
            pltpu.async_copy(
                idx_hbm.at[pl.ds((b * 3 + k) * N + ci * CHUNK, ln)],
                buf.at[pl.ds(k * CHUNK, ln)], sem)
            for k in range(3)
        ]

    def do_pair(pair, _):
        c0 = cg * ROWS_PER_WORKER + 2 * pair
        rw = [pltpu.async_copy(st_hbm.at[pl.ds((b * C + c0) * N, N)],
                               row_a, sem_row),
              pltpu.async_copy(st_hbm.at[pl.ds((b * C + c0 + 1) * N, N)],
                               row_b, sem_row)]
        pending_idx = start_idx(0, 0)
        for h in rw:
            h.wait()
        pending_out = [None, None]
        for ci in range(NCHUNK):
            s = ci % 2
            cur_idx = pending_idx
            if ci + 1 < NCHUNK:
                pending_idx = start_idx(ci + 1, (ci + 1) % 2)
            for h in cur_idx:
                h.wait()
            if pending_out[s] is not None:
                for h in pending_out[s]:
                    h.wait()
            base = ci * CHUNK
            ibuf, _ = idx_sets[s]
            obuf_a, obuf_b, sem_o = out_sets[s]

            def gather16(row, off):
                i0 = ibuf[pl.ds(off, 16)]
                i1 = ibuf[pl.ds(CHUNK + off, 16)]
                i2 = ibuf[pl.ds(2 * CHUNK + off, 16)]
                g = plsc.load_gather(row, [i0])
                g = jnp.maximum(g, plsc.load_gather(row, [i1]))
                g = jnp.maximum(g, plsc.load_gather(row, [i2]))
                return jnp.maximum(g, row[pl.ds(base + off, 16)])

            n32 = (CHUNK if ci < NCHUNK - 1 else TAIL - 16) // 32

            @plsc.parallel_loop(0, n32, unroll=4)
            def _(t, gather16=gather16, obuf_a=obuf_a, obuf_b=obuf_b):
                off = t * 32
                for row, obuf in ((row_a, obuf_a), (row_b, obuf_b)):
                    g_lo = gather16(row, off)
                    g_hi = gather16(row, off + 16)
                    obuf[pl.ds(off, 32)] = plsc.pack(
                        g_lo, g_hi, format=plsc.PackFormat.INTERLEAVED)

            if ci == NCHUNK - 1:
                for row, obuf in ((row_a, obuf_a), (row_b, obuf_b)):
                    g_lo = gather16(row, TAIL - 16)
                    obuf[pl.ds(TAIL - 16, 32)] = plsc.pack(
                        g_lo, g_lo, format=plsc.PackFormat.INTERLEAVED)

            ln = CHUNK if ci < NCHUNK - 1 else TAIL_DMA
            pending_out[s] = [
                pltpu.async_copy(
                    obuf_a.at[pl.ds(0, ln)],
                    out_hbm.at[pl.ds((b * C + c0) * NPAD + base, ln)], sem_o),
                pltpu.async_copy(
                    obuf_b.at[pl.ds(0, ln)],
                    out_hbm.at[pl.ds((b * C + c0 + 1) * NPAD + base, ln)],
                    sem_o)]
        for po in pending_out:
            if po is not None:
                for h in po:
                    h.wait()
        return 0

    lax.fori_loop(0, ROWS_PER_WORKER // 2, do_pair, 0)


def _sc_gather_max(st_flat, idx_flat):
    mesh = plsc.VectorSubcoreMesh(core_axis_name="c", subcore_axis_name="s")
    fn = pl.kernel(
        _sc_gather_max_body,
        out_type=jax.ShapeDtypeStruct((B * C * NPAD,), jnp.bfloat16),
        mesh=mesh,
        compiler_params=pltpu.CompilerParams(needs_layout_passes=False),
        scratch_types=[
            pltpu.VMEM((N,), jnp.float32),
            pltpu.VMEM((N,), jnp.float32),
            pltpu.VMEM((3 * CHUNK,), jnp.int32),
            pltpu.VMEM((3 * CHUNK,), jnp.int32),
            pltpu.VMEM((CHUNK + 16,), jnp.bfloat16),
            pltpu.VMEM((CHUNK + 16,), jnp.bfloat16),
            pltpu.VMEM((CHUNK + 16,), jnp.bfloat16),
            pltpu.VMEM((CHUNK + 16,), jnp.bfloat16),
            pltpu.SemaphoreType.DMA,
            pltpu.SemaphoreType.DMA,
            pltpu.SemaphoreType.DMA,
            pltpu.SemaphoreType.DMA,
            pltpu.SemaphoreType.DMA,
        ],
    )
    return fn(st_flat, idx_flat)


def _masked_psums(v, ni):
    lane = lax.broadcasted_iota(jnp.int32, (C, BLK), 1)
    valid = (lane + ni * BLK) < N
    vm = jnp.where(valid, v, 0.0)
    acc1 = jnp.zeros((C, 128), jnp.float32)
    acc2 = jnp.zeros((C, 128), jnp.float32)
    for j in range(BLK // 128):
        chunk = vm[:, j * 128:(j + 1) * 128]
        acc1 = acc1 + chunk
        acc2 = acc2 + chunk * chunk
    return acc1, acc2


def _k1_body(sp_ref, st_ref, w1_ref, w2_ref, b_ref, y_ref, psum_ref):
    bi = pl.program_id(0)
    ni = pl.program_id(1)
    y = jnp.dot(w1_ref[...], sp_ref[0], preferred_element_type=jnp.float32)
    y = y + jnp.dot(w2_ref[...], st_ref[0], preferred_element_type=jnp.float32)
    y = y + b_ref[...]
    y_ref[0] = y.astype(jnp.bfloat16)

    @pl.when((bi == 0) & (ni == 0))
    def _():
        psum_ref[...] = jnp.zeros_like(psum_ref)

    acc1, acc2 = _masked_psums(y, ni)
    psum_ref[0] += acc1
    psum_ref[1] += acc2


def _k1(spatial, structural, w1, w2, bias):
    return pl.pallas_call(
        _k1_body,
        grid=(B, NB),
        in_specs=[
            pl.BlockSpec((1, C, BLK), lambda b, n: (b, 0, n)),
            pl.BlockSpec((1, C, BLK), lambda b, n: (b, 0, n)),
            pl.BlockSpec((C, C), lambda b, n: (0, 0)),
            pl.BlockSpec((C, C), lambda b, n: (0, 0)),
            pl.BlockSpec((C, 1), lambda b, n: (0, 0)),
        ],
        out_specs=[
            pl.BlockSpec((1, C, BLK), lambda b, n: (b, 0, n)),
            pl.BlockSpec((2, C, 128), lambda b, n: (0, 0, 0)),
        ],
        out_shape=[
            jax.ShapeDtypeStruct((B, C, N), jnp.bfloat16),
            jax.ShapeDtypeStruct((2, C, 128), jnp.float32),
        ],
    )(spatial, structural, w1, w2, bias)


def _k23_body(y_ref, s_ref, sy_ref, oy_ref, w_ref, b_ref, g_ref, be_ref,
              out1_ref, out2_ref, psum_ref, sz_ref, oz_ref):
    p = pl.program_id(0)
    bi = pl.program_id(1)
    ni = pl.program_id(2)

    @pl.when(p == 0)
    def _():
        @pl.when((bi == 0) & (ni == 0))
        def _():
            psum_ref[...] = jnp.zeros_like(psum_ref)

        z = jnp.dot(w_ref[...], s_ref[0].astype(jnp.float32),
                    preferred_element_type=jnp.float32) + b_ref[...]
        acc1, acc2 = _masked_psums(z, ni)
        psum_ref[0] += acc1
        psum_ref[1] += acc2

    @pl.when(p == 1)
    def _():
        @pl.when((bi == 0) & (ni == 0))
        def _():
            tot = jnp.sum(psum_ref[...], axis=2)
            cnt = float(B * N)
            mean = tot[0] / cnt
            var = tot[1] / cnt - mean * mean
            scale = g_ref[0] * lax.rsqrt(var + EPS)
            shift = be_ref[0] - mean * scale
            sz_ref[...] = scale.reshape(C, 1)
            oz_ref[...] = shift.reshape(C, 1)

        y32 = y_ref[0].astype(jnp.float32)
        out1_ref[0] = jnp.maximum(y32 * sy_ref[...] + oy_ref[...], 0.0)
        z = jnp.dot(w_ref[...], s_ref[0].astype(jnp.float32),
                    preferred_element_type=jnp.float32) + b_ref[...]
        out2_ref[0] = jnp.maximum(z * sz_ref[...] + oz_ref[...], 0.0)


def _k23(y, s, sy, oy, w, bias, gamma, beta):
    return pl.pallas_call(
        _k23_body,
        grid=(2, B, NB),
        in_specs=[
            pl.BlockSpec((1, C, BLK), lambda p, b, n: (b * p, 0, n * p)),
            pl.BlockSpec((1, C, BLK), lambda p, b, n: (b, 0, n)),
            pl.BlockSpec((C, 1), lambda p, b, n: (0, 0)),
            pl.BlockSpec((C, 1), lambda p, b, n: (0, 0)),
            pl.BlockSpec((C, C), lambda p, b, n: (0, 0)),
            pl.BlockSpec((C, 1), lambda p, b, n: (0, 0)),
            pl.BlockSpec((1, C), lambda p, b, n: (0, 0)),
            pl.BlockSpec((1, C), lambda p, b, n: (0, 0)),
        ],
        out_specs=[
            pl.BlockSpec((1, C, BLK), lambda p, b, n: (b * p, 0, n * p)),
            pl.BlockSpec((1, C, BLK), lambda p, b, n: (b * p, 0, n * p)),
        ],
        out_shape=[
            jax.ShapeDtypeStruct((B, C, N), jnp.float32),
            jax.ShapeDtypeStruct((B, C, N), jnp.float32),
        ],
        scratch_shapes=[
            pltpu.VMEM((2, C, 128), jnp.float32),
            pltpu.VMEM((C, 1), jnp.float32),
            pltpu.VMEM((C, 1), jnp.float32),
        ],
    )(y, s, sy, oy, w, bias, gamma, beta)


def _bn_scale_shift(psum, gamma, beta):
    total = psum.sum(axis=2)
    cnt = float(B * N)
    mean = total[0] / cnt
    var = total[1] / cnt - mean * mean
    scale = gamma * lax.rsqrt(var + EPS)
    shift = beta - mean * scale
    return scale.reshape(C, 1), shift.reshape(C, 1)


def kernel(spatial_feat, structural_feat, neighbor_idx, comb_W, comb_b,
           comb_gamma, comb_beta, agg_W, agg_b, agg_gamma, agg_beta):
    st_flat = structural_feat.reshape(-1)
    idx_flat = jnp.transpose(neighbor_idx, (0, 2, 1)).reshape(-1)

    s_flat = _sc_gather_max(st_flat, idx_flat)
    s = s_flat.reshape(B, C, NPAD)

    w1 = comb_W[:, :C]
    w2 = comb_W[:, C:]
    y, psum_y = _k1(spatial_feat, structural_feat, w1, w2, comb_b.reshape(C, 1))

    sy, oy = _bn_scale_shift(psum_y, comb_gamma, comb_beta)

    out1, out2 = _k23(y, s, sy, oy, agg_W, agg_b.reshape(C, 1),
                      agg_gamma.reshape(1, C), agg_beta.reshape(1, C))
    return (out1, out2)
